# baseline (speedup 1.0000x reference)
import jax
import jax.numpy as jnp
from jax import lax
from jax.experimental import pallas as pl
from jax.experimental.pallas import tpu as pltpu
from jax.experimental.pallas import tpu_sc as plsc

E = 8
TOP_K = 2
H = 1024
F = 1024
T = 2048
A = TOP_K * T
RBLK = 256
NBLK = A // RBLK + E
NROWS = NBLK * RBLK
LANES = 128
NC = 2
NS = 16
NW = NC * NS
APW = A // NS

_sc_mesh = plsc.VectorSubcoreMesh(
    core_axis_name="c", subcore_axis_name="s", num_cores=NC, num_subcores=NS
)


def _lane16():
    return lax.broadcasted_iota(jnp.int32, (16,), 0)


def _router_body(x_ref, rw_ref, eids_ref, wts_ref):
    rw = rw_ref[...]
    x = x_ref[...]
    logits = lax.dot_general(
        rw, x, (((1,), (1,)), ((), ())), preferred_element_type=jnp.float32
    )
    row = lax.broadcasted_iota(jnp.int32, logits.shape, 0)
    neg = jnp.float32(-1e30)
    logits = jnp.where(row < E, logits, neg)
    m = jnp.max(logits, axis=0, keepdims=True)
    ex = jnp.exp(logits - m)
    ex = jnp.where(row < E, ex, 0.0)
    scores = ex / jnp.sum(ex, axis=0, keepdims=True)
    big = jnp.int32(LANES)
    m1 = jnp.max(scores, axis=0, keepdims=True)
    i1 = jnp.min(jnp.where(scores == m1, row, big), axis=0, keepdims=True)
    sc2 = jnp.where(row == i1, neg, scores)
    m2 = jnp.max(sc2, axis=0, keepdims=True)
    i2 = jnp.min(jnp.where(sc2 == m2, row, big), axis=0, keepdims=True)
    denom = m1 + m2
    krow = lax.broadcasted_iota(jnp.int32, (E, T), 0)
    eids_ref[...] = jnp.where(
        krow == 0, jnp.broadcast_to(i1, (E, T)),
        jnp.where(krow == 1, jnp.broadcast_to(i2, (E, T)), 0),
    )
    wts_ref[...] = jnp.where(
        krow == 0, jnp.broadcast_to(m1 / denom, (E, T)),
        jnp.where(krow == 1, jnp.broadcast_to(m2 / denom, (E, T)), 0.0),
    )


def _router(xf, rw_pad):
    return pl.pallas_call(
        _router_body,
        grid=(1,),
        in_specs=[
            pl.BlockSpec((T, H), lambda i: (0, 0)),
            pl.BlockSpec((LANES, H), lambda i: (0, 0)),
        ],
        out_specs=[
            pl.BlockSpec((E, T), lambda i: (0, 0)),
            pl.BlockSpec((E, T), lambda i: (0, 0)),
        ],
        out_shape=[
            jax.ShapeDtypeStruct((E, T), jnp.int32),
            jax.ShapeDtypeStruct((E, T), jnp.float32),
        ],
    )(xf, rw_pad)


def _meta_body(eids_hbm, sortpos_hbm, stok_hbm, bexp_hbm, nrows_hbm,
               ev, pA, pB, tA, tB, histb, hall, bexb, nrb, zi, shist, stok):
    cid = lax.axis_index("c")
    sid = lax.axis_index("s")
    lane = _lane16()
    zeros16 = jnp.zeros((16,), jnp.int32)

    @pl.when(cid == 0)
    def _():
        for j in range(NROWS // NS // 16):
            zi[pl.ds(j * 16, 16)] = zeros16
        pltpu.sync_copy(zi, stok.at[pl.ds(sid * (NROWS // NS), NROWS // NS)])

    plsc.subcore_barrier()

    @pl.when(cid == 0)
    def _():
        pltpu.sync_copy(eids_hbm.at[pl.ds(sid * APW, APW)], ev)
        cnts = []
        for e in range(E):
            acc = jnp.int32(0)
            for j in range(APW // 16):
                evj = ev[pl.ds(j * 16, 16)]
                acc = acc + jnp.sum(jnp.where(evj == e, 1, 0))
            cnts.append(acc)
        hv = zeros16
        for e in range(E):
            hv = hv + jnp.where(lane == e, cnts[e], 0)
        histb[...] = hv
        pltpu.sync_copy(histb, shist.at[sid])

    plsc.subcore_barrier()

    @pl.when(cid == 0)
    def _():
        pltpu.sync_copy(shist, hall)
        total = zeros16
        base = zeros16
        for w in range(NS):
            hw = hall[w]
            total = total + hw
            base = base + jnp.where(jnp.int32(w) < sid, hw, 0)
        padded = ((total + (RBLK - 1)) >> 8) << 8
        ex_off = plsc.cumsum(padded) - padded
        start = ex_off + base

        cnt = []
        for e in range(E):
            cnt.append(jnp.sum(jnp.where(lane == e, start, 0)))

        @pl.when(sid == 0)
        def _():
            nr = jnp.sum(padded)
            nrb[...] = jnp.full((16,), nr, jnp.int32)
            pltpu.sync_copy(nrb, nrows_hbm)
            off_s = [jnp.sum(jnp.where(lane == e, ex_off, 0)) for e in range(E)]
            pad_s = [jnp.sum(jnp.where(lane == e, padded, 0)) for e in range(E)]
            last_e = jnp.max(jnp.where(padded > 0, lane, 0))
            for v in range(2):
                b = lane + v * 16
                r0 = b * RBLK
                bx = zeros16
                for e in range(E):
                    inside = (r0 >= off_s[e]) & (r0 < off_s[e] + pad_s[e])
                    bx = bx + jnp.where(inside, e, 0)
                bx = jnp.where(r0 < nr, bx, last_e)
                bexb[pl.ds(v * 16, 16)] = bx
            pltpu.sync_copy(bexb, bexp_hbm)

        for j in range(APW // 16):
            evj = ev[pl.ds(j * 16, 16)]
            pos = zeros16
            for e in range(E):
                m = evj == e
                mi = jnp.where(m, 1, 0)
                pref = plsc.cumsum(mi) - mi
                pos = jnp.where(m, cnt[e] + pref, pos)
                cnt[e] = cnt[e] + jnp.sum(mi)
            pbuf, tbuf = (pA, tA) if j < 8 else (pB, tB)
            off = (j % 8) * 16
            pbuf[pl.ds(off, 16)] = pos
            gi = sid * APW + j * 16 + lane
            tbuf[pl.ds(off, 16)] = gi & (T - 1)
        pltpu.sync_copy(pA, sortpos_hbm.at[pl.ds(sid * APW, 128)])
        pltpu.sync_copy(pB, sortpos_hbm.at[pl.ds(sid * APW + 128, 128)])

        pltpu.sync_copy(tA, stok.at[pA])
        pltpu.sync_copy(tB, stok.at[pB])

    plsc.subcore_barrier()

    @pl.when(cid == 0)
    def _():
        pltpu.sync_copy(stok.at[pl.ds(sid * (NROWS // NS), NROWS // NS)], zi)
        pltpu.sync_copy(zi, stok_hbm.at[pl.ds(sid * (NROWS // NS), NROWS // NS)])


_meta = pl.kernel(
    _meta_body,
    out_type=[
        jax.ShapeDtypeStruct((A,), jnp.int32),
        jax.ShapeDtypeStruct((NROWS,), jnp.int32),
        jax.ShapeDtypeStruct((32,), jnp.int32),
        jax.ShapeDtypeStruct((16,), jnp.int32),
    ],
    mesh=_sc_mesh,
    compiler_params=pltpu.CompilerParams(needs_layout_passes=False),
    scratch_types=[
        pltpu.VMEM((APW,), jnp.int32),
        pltpu.VMEM((128,), jnp.int32),
        pltpu.VMEM((128,), jnp.int32),
        pltpu.VMEM((128,), jnp.int32),
        pltpu.VMEM((128,), jnp.int32),
        pltpu.VMEM((16,), jnp.int32),
        pltpu.VMEM((NS, 16), jnp.int32),
        pltpu.VMEM((32,), jnp.int32),
        pltpu.VMEM((16,), jnp.int32),
        pltpu.VMEM((NROWS // NS,), jnp.int32),
        pltpu.VMEM_SHARED((NS, 16), jnp.int32),
        pltpu.VMEM_SHARED((NROWS,), jnp.int32),
    ],
)


GCH = 64


def _gather_body(x_hbm, stok_hbm, xs_hbm, idxb, buf, sem):
    cid = lax.axis_index("c")
    sid = lax.axis_index("s")
    wid = sid * NC + cid
    for j in range(NROWS // GCH // NW):
        base = (wid + j * NW) * GCH
        pltpu.sync_copy(stok_hbm.at[pl.ds(base, GCH)], idxb)
        pltpu.async_copy(x_hbm.at[idxb], buf, sem).wait()
        pltpu.sync_copy(buf, xs_hbm.at[pl.ds(base, GCH)])


_gather = pl.kernel(
    _gather_body,
    out_type=jax.ShapeDtypeStruct((NROWS, H), jnp.float32),
    mesh=_sc_mesh,
    compiler_params=pltpu.CompilerParams(needs_layout_passes=False),
    scratch_types=[
        pltpu.VMEM((GCH,), jnp.int32),
        pltpu.VMEM((GCH, H), jnp.float32),
        pltpu.SemaphoreType.DMA,
    ],
)


def _gemm_body(bexp_ref, xs_ref, w1_ref, v1_ref, w2_ref, y_ref):
    xb = xs_ref[...].astype(jnp.bfloat16)
    w1b = w1_ref[0].astype(jnp.bfloat16)
    v1b = v1_ref[0].astype(jnp.bfloat16)
    w2b = w2_ref[0].astype(jnp.bfloat16)
    h1 = lax.dot_general(
        xb, w1b, (((1,), (1,)), ((), ())), preferred_element_type=jnp.float32
    )
    h2 = lax.dot_general(
        xb, v1b, (((1,), (1,)), ((), ())), preferred_element_type=jnp.float32
    )
    h = (h1 * jax.nn.sigmoid(h1) * h2).astype(jnp.bfloat16)
    y_ref[...] = lax.dot_general(
        h, w2b, (((1,), (0,)), ((), ())), preferred_element_type=jnp.float32
    )


def _gemm(bexp, xs, w1, v1, w2):
    grid_spec = pltpu.PrefetchScalarGridSpec(
        num_scalar_prefetch=1,
        grid=(NBLK,),
        in_specs=[
            pl.BlockSpec((RBLK, H), lambda b, be: (b, 0)),
            pl.BlockSpec((1, F, H), lambda b, be: (be[b], 0, 0)),
            pl.BlockSpec((1, F, H), lambda b, be: (be[b], 0, 0)),
            pl.BlockSpec((1, F, H), lambda b, be: (be[b], 0, 0)),
        ],
        out_specs=pl.BlockSpec((RBLK, H), lambda b, be: (b, 0)),
    )
    return pl.pallas_call(
        _gemm_body,
        grid_spec=grid_spec,
        out_shape=jax.ShapeDtypeStruct((NROWS, H), jnp.float32),
    )(bexp, xs, w1, v1, w2)


CCH = 16


def _combine_body(y_hbm, sortpos_hbm, wts_hbm, out_hbm,
                  i0, i1, w0b, w1b, buf0, buf1, ob, sem):
    cid = lax.axis_index("c")
    sid = lax.axis_index("s")
    wid = sid * NC + cid
    lane = _lane16()
    for q in range(T // CCH // NW):
        t0 = (wid + q * NW) * CCH
        pltpu.sync_copy(sortpos_hbm.at[pl.ds(t0, CCH)], i0)
        pltpu.sync_copy(sortpos_hbm.at[pl.ds(T + t0, CCH)], i1)
        pltpu.sync_copy(wts_hbm.at[0, pl.ds(t0, CCH)], w0b)
        pltpu.sync_copy(wts_hbm.at[1, pl.ds(t0, CCH)], w1b)
        cp0 = pltpu.async_copy(y_hbm.at[i0], buf0, sem)
        cp1 = pltpu.async_copy(y_hbm.at[i1], buf1, sem)
        cp0.wait()
        cp1.wait()
        w0v = w0b[...]
        w1v = w1b[...]
        for i in range(CCH):
            s0 = jnp.sum(jnp.where(lane == i, w0v, 0.0))
            s1 = jnp.sum(jnp.where(lane == i, w1v, 0.0))

            def body(c, _):
                sl = pl.ds(c * 16, 16)
                ob[i, sl] = buf0[i, sl] * s0 + buf1[i, sl] * s1
                return 0

            lax.fori_loop(0, H // 16, body, 0)
        pltpu.sync_copy(ob, out_hbm.at[pl.ds(t0, CCH)])


_combine = pl.kernel(
    _combine_body,
    out_type=jax.ShapeDtypeStruct((T, H), jnp.float32),
    mesh=_sc_mesh,
    compiler_params=pltpu.CompilerParams(needs_layout_passes=False),
    scratch_types=[
        pltpu.VMEM((CCH,), jnp.int32),
        pltpu.VMEM((CCH,), jnp.int32),
        pltpu.VMEM((CCH,), jnp.float32),
        pltpu.VMEM((CCH,), jnp.float32),
        pltpu.VMEM((CCH, H), jnp.float32),
        pltpu.VMEM((CCH, H), jnp.float32),
        pltpu.VMEM((CCH, H), jnp.float32),
        pltpu.SemaphoreType.DMA,
    ],
)


_SC_META = False
_SC_GATHER = True
_SC_COMBINE = True


def _meta_jnp(eids_flat):
    cnt = jnp.bincount(eids_flat, length=E)
    padded = ((cnt + (RBLK - 1)) // RBLK) * RBLK
    poff = jnp.cumsum(padded) - padded
    coff = jnp.cumsum(cnt) - cnt
    order = jnp.argsort(eids_flat, stable=True)
    e_of = eids_flat[order]
    ranks = jnp.arange(A, dtype=jnp.int32)
    pos_of_order = poff[e_of] + ranks - coff[e_of]
    sortpos = jnp.zeros((A,), jnp.int32).at[order].set(pos_of_order)
    sorted_tok = jnp.zeros((NROWS,), jnp.int32).at[pos_of_order].set(order & (T - 1))
    nr = jnp.sum(padded)
    blk = jnp.arange(NBLK, dtype=jnp.int32) * RBLK
    last_e = jnp.max(jnp.where(padded > 0, jnp.arange(E), 0)).astype(jnp.int32)
    inside = (blk[:, None] >= poff[None, :]) & (blk[:, None] < (poff + padded)[None, :])
    bexp = jnp.sum(jnp.where(inside, jnp.arange(E)[None, :], 0), axis=1).astype(jnp.int32)
    bexp = jnp.where(blk < nr, bexp, last_e)
    return sortpos, sorted_tok, bexp, None


@jax.jit
def kernel(hidden_states, router_w, w1, v1, w2):
    xf = hidden_states.reshape(T, H)
    rw_pad = jnp.zeros((LANES, H), jnp.float32).at[:E].set(router_w)

    eids, wts = _router(xf, rw_pad)
    eids_flat = eids[:TOP_K].reshape(A)
    if _SC_META:
        sortpos, sorted_tok, bexp, nrows = _meta(eids_flat)
        bexp = bexp[:NBLK]
    else:
        sortpos, sorted_tok, bexp, _ = _meta_jnp(eids_flat)
    if _SC_GATHER:
        xs = _gather(xf, sorted_tok)
    else:
        xs = xf[sorted_tok]
    y = _gemm(bexp, xs, w1, v1, w2)
    if _SC_COMBINE:
        out = _combine(y, sortpos, wts[:TOP_K])
    else:
        w2d = wts[:TOP_K]
        out = w2d[0][:, None] * y[sortpos[:T]] + w2d[1][:, None] * y[sortpos[T:]]
    return out.reshape(1, T, H)


# speedup vs baseline: 1.5376x; 1.5376x over previous
import jax
import jax.numpy as jnp
from jax import lax
from jax.experimental import pallas as pl
from jax.experimental.pallas import tpu as pltpu
from jax.experimental.pallas import tpu_sc as plsc

E = 8
TOP_K = 2
H = 1024
F = 1024
T = 2048
A = TOP_K * T
RBLK = 256
NBLK = A // RBLK + E
NROWS = NBLK * RBLK
LANES = 128
NC = 2
NS = 16
NW = NC * NS
APW = A // NW

_sc_mesh = plsc.VectorSubcoreMesh(
    core_axis_name="c", subcore_axis_name="s", num_cores=NC, num_subcores=NS
)
_sc_params = pltpu.CompilerParams(needs_layout_passes=False)


def _lane16():
    return lax.broadcasted_iota(jnp.int32, (16,), 0)


def _router_body(x_ref, rw_ref, eids_ref, wts_ref, hist_ref):
    rw = rw_ref[...]
    x = x_ref[...]
    logits = lax.dot_general(
        rw, x, (((1,), (1,)), ((), ())), preferred_element_type=jnp.float32
    )
    row = lax.broadcasted_iota(jnp.int32, logits.shape, 0)
    neg = jnp.float32(-1e30)
    logits = jnp.where(row < E, logits, neg)
    m = jnp.max(logits, axis=0, keepdims=True)
    ex = jnp.exp(logits - m)
    ex = jnp.where(row < E, ex, 0.0)
    scores = ex / jnp.sum(ex, axis=0, keepdims=True)
    big = jnp.int32(LANES)
    m1 = jnp.max(scores, axis=0, keepdims=True)
    i1 = jnp.min(jnp.where(scores == m1, row, big), axis=0, keepdims=True)
    sc2 = jnp.where(row == i1, neg, scores)
    m2 = jnp.max(sc2, axis=0, keepdims=True)
    i2 = jnp.min(jnp.where(sc2 == m2, row, big), axis=0, keepdims=True)
    denom = m1 + m2
    krow = lax.broadcasted_iota(jnp.int32, (E, T), 0)
    eids_ref[...] = jnp.where(
        krow == 0, jnp.broadcast_to(i1, (E, T)),
        jnp.where(krow == 1, jnp.broadcast_to(i2, (E, T)), 0),
    )
    wts_ref[...] = jnp.where(
        krow == 0, jnp.broadcast_to(m1 / denom, (E, T)),
        jnp.where(krow == 1, jnp.broadcast_to(m2 / denom, (E, T)), 0.0),
    )
    erow = lax.broadcasted_iota(jnp.int32, (E, T), 0)
    oh0 = (erow == jnp.broadcast_to(i1, (E, T))).astype(jnp.float32)
    oh1 = (erow == jnp.broadcast_to(i2, (E, T))).astype(jnp.float32)
    tw = lax.broadcasted_iota(jnp.int32, (T, NS), 0) // APW
    ww = lax.broadcasted_iota(jnp.int32, (T, NS), 1)
    sel = (tw == ww).astype(jnp.float32)
    h0 = lax.dot_general(sel, oh0, (((0,), (1,)), ((), ())),
                         preferred_element_type=jnp.float32)
    h1 = lax.dot_general(sel, oh1, (((0,), (1,)), ((), ())),
                         preferred_element_type=jnp.float32)
    h01 = jnp.concatenate([h0, h1], axis=0).astype(jnp.int32)
    hist_ref[...] = jnp.concatenate(
        [h01, jnp.zeros((NW, LANES - E), jnp.int32)], axis=1
    )


def _router(xf, rw_pad):
    return pl.pallas_call(
        _router_body,
        grid=(1,),
        in_specs=[
            pl.BlockSpec((T, H), lambda i: (0, 0)),
            pl.BlockSpec((LANES, H), lambda i: (0, 0)),
        ],
        out_specs=[
            pl.BlockSpec((E, T), lambda i: (0, 0)),
            pl.BlockSpec((E, T), lambda i: (0, 0)),
            pl.BlockSpec((NW, LANES), lambda i: (0, 0)),
        ],
        out_shape=[
            jax.ShapeDtypeStruct((E, T), jnp.int32),
            jax.ShapeDtypeStruct((E, T), jnp.float32),
            jax.ShapeDtypeStruct((NW, LANES), jnp.int32),
        ],
    )(xf, rw_pad)


def _dispatch_body(eids_hbm, hist_hbm, sortpos_hbm, stok_hbm, bexp_hbm,
                   ev, hb, posb, tokb, bexb):
    cid = lax.axis_index("c")
    sid = lax.axis_index("s")
    wid = sid * NC + cid
    lane = _lane16()
    zeros16 = jnp.zeros((16,), jnp.int32)

    pltpu.sync_copy(eids_hbm.at[pl.ds(wid * APW, APW)], ev)
    pltpu.sync_copy(hist_hbm, hb)

    total = zeros16
    base = zeros16
    for w in range(NW):
        hw = hb[w, pl.ds(0, 16)]
        total = total + hw
        base = base + jnp.where(jnp.int32(w) < wid, hw, 0)
    padded = ((total + (RBLK - 1)) >> 8) << 8
    ex_off = plsc.cumsum(padded) - padded
    start = ex_off + base

    cnt = []
    for e in range(E):
        cnt.append(jnp.sum(jnp.where(lane == e, start, 0)))

    nr = jnp.sum(padded)
    off_s = [jnp.sum(jnp.where(lane == e, ex_off, 0)) for e in range(E)]
    pad_s = [jnp.sum(jnp.where(lane == e, padded, 0)) for e in range(E)]
    last_e = jnp.max(jnp.where(padded > 0, lane, 0))
    for v in range(2):
        b = lane + v * 16
        r0 = b * RBLK
        bx = zeros16
        for e in range(E):
            inside = (r0 >= off_s[e]) & (r0 < off_s[e] + pad_s[e])
            bx = bx + jnp.where(inside, e, 0)
        bx = jnp.where(r0 < nr, bx, last_e)
        bexb[pl.ds(v * 16, 16)] = bx
    pltpu.sync_copy(bexb, bexp_hbm.at[wid])

    for j in range(APW // 16):
        evj = ev[pl.ds(j * 16, 16)]
        pos = zeros16
        for e in range(E):
            mask = evj == e
            mi = jnp.where(mask, 1, 0)
            pref = plsc.cumsum(mi) - mi
            pos = jnp.where(mask, cnt[e] + pref, pos)
            cnt[e] = cnt[e] + jnp.sum(mi)
        posb[pl.ds(j * 16, 16)] = pos
        gi = wid * APW + j * 16 + lane
        tokb[pl.ds(j * 16, 16)] = gi & (T - 1)
    pltpu.sync_copy(posb, sortpos_hbm.at[pl.ds(wid * APW, APW)])
    pltpu.sync_copy(tokb, stok_hbm.at[posb])


_dispatch = pl.kernel(
    _dispatch_body,
    out_type=[
        jax.ShapeDtypeStruct((A,), jnp.int32),
        jax.ShapeDtypeStruct((NROWS,), jnp.int32),
        jax.ShapeDtypeStruct((NW, 32), jnp.int32),
    ],
    mesh=_sc_mesh,
    compiler_params=_sc_params,
    scratch_types=[
        pltpu.VMEM((APW,), jnp.int32),
        pltpu.VMEM((NW, LANES), jnp.int32),
        pltpu.VMEM((APW,), jnp.int32),
        pltpu.VMEM((APW,), jnp.int32),
        pltpu.VMEM((32,), jnp.int32),
    ],
)


GCH = 64


def _gather_body(x_hbm, stok_hbm, xs_hbm, idxb, buf, sem):
    cid = lax.axis_index("c")
    sid = lax.axis_index("s")
    wid = sid * NC + cid
    for j in range(NROWS // GCH // NW):
        base = (wid + j * NW) * GCH
        pltpu.sync_copy(stok_hbm.at[pl.ds(base, GCH)], idxb)
        for q in range(GCH // 16):
            sl = pl.ds(q * 16, 16)
            idxb[sl] = idxb[sl] & (T - 1)
        pltpu.async_copy(x_hbm.at[idxb], buf, sem).wait()
        pltpu.sync_copy(buf, xs_hbm.at[pl.ds(base, GCH)])


_gather = pl.kernel(
    _gather_body,
    out_type=jax.ShapeDtypeStruct((NROWS, H), jnp.float32),
    mesh=_sc_mesh,
    compiler_params=_sc_params,
    scratch_types=[
        pltpu.VMEM((GCH,), jnp.int32),
        pltpu.VMEM((GCH, H), jnp.float32),
        pltpu.SemaphoreType.DMA,
    ],
)


def _gemm_body(bexp_ref, xs_ref, w1_ref, v1_ref, w2_ref, y_ref):
    xb = xs_ref[...].astype(jnp.bfloat16)
    w1b = w1_ref[0].astype(jnp.bfloat16)
    v1b = v1_ref[0].astype(jnp.bfloat16)
    w2b = w2_ref[0].astype(jnp.bfloat16)
    h1 = lax.dot_general(
        xb, w1b, (((1,), (1,)), ((), ())), preferred_element_type=jnp.float32
    )
    h2 = lax.dot_general(
        xb, v1b, (((1,), (1,)), ((), ())), preferred_element_type=jnp.float32
    )
    h = (h1 * jax.nn.sigmoid(h1) * h2).astype(jnp.bfloat16)
    y_ref[...] = lax.dot_general(
        h, w2b, (((1,), (0,)), ((), ())), preferred_element_type=jnp.float32
    )


def _gemm(bexp, xs, w1, v1, w2):
    grid_spec = pltpu.PrefetchScalarGridSpec(
        num_scalar_prefetch=1,
        grid=(NBLK,),
        in_specs=[
            pl.BlockSpec((RBLK, H), lambda b, be: (b, 0)),
            pl.BlockSpec((1, F, H), lambda b, be: (be[b], 0, 0)),
            pl.BlockSpec((1, F, H), lambda b, be: (be[b], 0, 0)),
            pl.BlockSpec((1, F, H), lambda b, be: (be[b], 0, 0)),
        ],
        out_specs=pl.BlockSpec((RBLK, H), lambda b, be: (b, 0)),
    )
    return pl.pallas_call(
        _gemm_body,
        grid_spec=grid_spec,
        out_shape=jax.ShapeDtypeStruct((NROWS, H), jnp.float32),
    )(bexp, xs, w1, v1, w2)


CCH = 16


def _combine_body(y_hbm, sortpos_hbm, wts_hbm, out_hbm,
                  i0, i1, w0b, w1b, buf0, buf1, ob, sem):
    cid = lax.axis_index("c")
    sid = lax.axis_index("s")
    wid = sid * NC + cid
    lane = _lane16()
    for q in range(T // CCH // NW):
        t0 = (wid + q * NW) * CCH
        pltpu.sync_copy(sortpos_hbm.at[pl.ds(t0, CCH)], i0)
        pltpu.sync_copy(sortpos_hbm.at[pl.ds(T + t0, CCH)], i1)
        pltpu.sync_copy(wts_hbm.at[0, pl.ds(t0, CCH)], w0b)
        pltpu.sync_copy(wts_hbm.at[1, pl.ds(t0, CCH)], w1b)
        cp0 = pltpu.async_copy(y_hbm.at[i0], buf0, sem)
        cp1 = pltpu.async_copy(y_hbm.at[i1], buf1, sem)
        cp0.wait()
        cp1.wait()
        w0v = w0b[...]
        w1v = w1b[...]
        for i in range(CCH):
            s0 = jnp.sum(jnp.where(lane == i, w0v, 0.0))
            s1 = jnp.sum(jnp.where(lane == i, w1v, 0.0))

            def body(c, _):
                sl = pl.ds(c * 16, 16)
                ob[i, sl] = buf0[i, sl] * s0 + buf1[i, sl] * s1
                return 0

            lax.fori_loop(0, H // 16, body, 0)
        pltpu.sync_copy(ob, out_hbm.at[pl.ds(t0, CCH)])


_combine = pl.kernel(
    _combine_body,
    out_type=jax.ShapeDtypeStruct((T, H), jnp.float32),
    mesh=_sc_mesh,
    compiler_params=_sc_params,
    scratch_types=[
        pltpu.VMEM((CCH,), jnp.int32),
        pltpu.VMEM((CCH,), jnp.int32),
        pltpu.VMEM((CCH,), jnp.float32),
        pltpu.VMEM((CCH,), jnp.float32),
        pltpu.VMEM((CCH, H), jnp.float32),
        pltpu.VMEM((CCH, H), jnp.float32),
        pltpu.VMEM((CCH, H), jnp.float32),
        pltpu.SemaphoreType.DMA,
    ],
)


@jax.jit
def kernel(hidden_states, router_w, w1, v1, w2):
    xf = hidden_states.reshape(T, H)
    rw_pad = jnp.zeros((LANES, H), jnp.float32).at[:E].set(router_w)

    eids, wts, hist = _router(xf, rw_pad)
    eids_flat = eids[:TOP_K].reshape(A)
    sortpos, sorted_tok, bexp = _dispatch(eids_flat, hist)
    xs = _gather(xf, sorted_tok)
    y = _gemm(bexp[0, :NBLK], xs, w1, v1, w2)
    out = _combine(y, sortpos, wts[:TOP_K])
    return out.reshape(1, T, H)


# speedup vs baseline: 1.7731x; 1.1532x over previous
import jax
import jax.numpy as jnp
from jax import lax
from jax.experimental import pallas as pl
from jax.experimental.pallas import tpu as pltpu
from jax.experimental.pallas import tpu_sc as plsc

E = 8
TOP_K = 2
H = 1024
F = 1024
T = 2048
A = TOP_K * T
RBLK = 256
NBLK = A // RBLK + E
NROWS = NBLK * RBLK
LANES = 128
NC = 2
NS = 16
NW = NC * NS
APW = A // NW

_sc_mesh = plsc.VectorSubcoreMesh(
    core_axis_name="c", subcore_axis_name="s", num_cores=NC, num_subcores=NS
)
_sc_params = pltpu.CompilerParams(needs_layout_passes=False)


def _lane16():
    return lax.broadcasted_iota(jnp.int32, (16,), 0)


def _router_body(x_ref, rw_ref, eids_ref, wts_ref, hist_ref):
    rw = rw_ref[...]
    x = x_ref[...]
    logits = lax.dot_general(
        rw, x, (((1,), (1,)), ((), ())), preferred_element_type=jnp.float32
    )
    row = lax.broadcasted_iota(jnp.int32, logits.shape, 0)
    neg = jnp.float32(-1e30)
    logits = jnp.where(row < E, logits, neg)
    m = jnp.max(logits, axis=0, keepdims=True)
    ex = jnp.exp(logits - m)
    ex = jnp.where(row < E, ex, 0.0)
    scores = ex / jnp.sum(ex, axis=0, keepdims=True)
    big = jnp.int32(LANES)
    m1 = jnp.max(scores, axis=0, keepdims=True)
    i1 = jnp.min(jnp.where(scores == m1, row, big), axis=0, keepdims=True)
    sc2 = jnp.where(row == i1, neg, scores)
    m2 = jnp.max(sc2, axis=0, keepdims=True)
    i2 = jnp.min(jnp.where(sc2 == m2, row, big), axis=0, keepdims=True)
    denom = m1 + m2
    krow = lax.broadcasted_iota(jnp.int32, (E, T), 0)
    eids_ref[...] = jnp.where(
        krow == 0, jnp.broadcast_to(i1, (E, T)),
        jnp.where(krow == 1, jnp.broadcast_to(i2, (E, T)), 0),
    )
    wts_ref[...] = jnp.where(
        krow == 0, jnp.broadcast_to(m1 / denom, (E, T)),
        jnp.where(krow == 1, jnp.broadcast_to(m2 / denom, (E, T)), 0.0),
    )
    erow = lax.broadcasted_iota(jnp.int32, (E, T), 0)
    oh0 = (erow == jnp.broadcast_to(i1, (E, T))).astype(jnp.float32)
    oh1 = (erow == jnp.broadcast_to(i2, (E, T))).astype(jnp.float32)
    tw = lax.broadcasted_iota(jnp.int32, (T, NS), 0) // APW
    ww = lax.broadcasted_iota(jnp.int32, (T, NS), 1)
    sel = (tw == ww).astype(jnp.float32)
    h0 = lax.dot_general(oh0, sel, (((1,), (0,)), ((), ())),
                         preferred_element_type=jnp.float32)
    h1 = lax.dot_general(oh1, sel, (((1,), (0,)), ((), ())),
                         preferred_element_type=jnp.float32)
    h01 = jnp.concatenate([h0, h1], axis=1).astype(jnp.int32)
    hist_ref[...] = jnp.concatenate(
        [h01, jnp.zeros((E, LANES - NW), jnp.int32)], axis=1
    )


def _router(xf, rw_pad):
    return pl.pallas_call(
        _router_body,
        grid=(1,),
        in_specs=[
            pl.BlockSpec((T, H), lambda i: (0, 0)),
            pl.BlockSpec((LANES, H), lambda i: (0, 0)),
        ],
        out_specs=[
            pl.BlockSpec((E, T), lambda i: (0, 0)),
            pl.BlockSpec((E, T), lambda i: (0, 0)),
            pl.BlockSpec((E, LANES), lambda i: (0, 0)),
        ],
        out_shape=[
            jax.ShapeDtypeStruct((E, T), jnp.int32),
            jax.ShapeDtypeStruct((E, T), jnp.float32),
            jax.ShapeDtypeStruct((E, LANES), jnp.int32),
        ],
    )(xf, rw_pad)


def _dispatch_body(eids_hbm, hist_hbm, sortpos_hbm, stok_hbm, bexp_hbm,
                   act_hbm, ev, hb, posb, tokb, bexb, actb):
    cid = lax.axis_index("c")
    sid = lax.axis_index("s")
    wid = sid * NC + cid
    lane = _lane16()
    zeros16 = jnp.zeros((16,), jnp.int32)

    pltpu.sync_copy(eids_hbm.at[pl.ds(wid * APW, APW)], ev)
    pltpu.sync_copy(hist_hbm, hb)

    tot_s, base_s = [], []
    for e in range(E):
        he0 = hb[e, pl.ds(0, 16)]
        he1 = hb[e, pl.ds(16, 16)]
        tot_s.append(jnp.sum(he0) + jnp.sum(he1))
        base_s.append(
            jnp.sum(jnp.where(lane < wid, he0, 0))
            + jnp.sum(jnp.where(lane + 16 < wid, he1, 0))
        )
    total = zeros16
    for e in range(E):
        total = total + jnp.where(lane == e, tot_s[e], 0)
    padded = ((total + (RBLK - 1)) >> 8) << 8
    ex_off = plsc.cumsum(padded) - padded

    off_s = [jnp.sum(jnp.where(lane == e, ex_off, 0)) for e in range(E)]
    pad_s = [jnp.sum(jnp.where(lane == e, padded, 0)) for e in range(E)]
    cnt = [off_s[e] + base_s[e] for e in range(E)]

    nr = jnp.sum(padded)
    last_e = jnp.max(jnp.where(padded > 0, lane, 0))
    for v in range(2):
        b = lane + v * 16
        r0 = b * RBLK
        bx = zeros16
        for e in range(E):
            inside = (r0 >= off_s[e]) & (r0 < off_s[e] + pad_s[e])
            bx = bx + jnp.where(inside, e, 0)
        active = r0 < nr
        bexb[pl.ds(v * 16, 16)] = jnp.where(active, bx, last_e)
        actb[pl.ds(v * 16, 16)] = jnp.where(active, 1, 0)
    pltpu.sync_copy(bexb, bexp_hbm.at[wid])
    pltpu.sync_copy(actb, act_hbm.at[wid])

    for j in range(APW // 16):
        evj = ev[pl.ds(j * 16, 16)]
        pos = zeros16
        for e in range(E):
            mask = evj == e
            mi = jnp.where(mask, 1, 0)
            pref = plsc.cumsum(mi) - mi
            pos = jnp.where(mask, cnt[e] + pref, pos)
            cnt[e] = cnt[e] + jnp.sum(mi)
        posb[pl.ds(j * 16, 16)] = pos
        gi = wid * APW + j * 16 + lane
        tokb[pl.ds(j * 16, 16)] = gi & (T - 1)
    pltpu.sync_copy(posb, sortpos_hbm.at[pl.ds(wid * APW, APW)])
    pltpu.sync_copy(tokb, stok_hbm.at[posb])


_dispatch = pl.kernel(
    _dispatch_body,
    out_type=[
        jax.ShapeDtypeStruct((A,), jnp.int32),
        jax.ShapeDtypeStruct((NROWS,), jnp.int32),
        jax.ShapeDtypeStruct((NW, 32), jnp.int32),
        jax.ShapeDtypeStruct((NW, 32), jnp.int32),
    ],
    mesh=_sc_mesh,
    compiler_params=_sc_params,
    scratch_types=[
        pltpu.VMEM((APW,), jnp.int32),
        pltpu.VMEM((E, LANES), jnp.int32),
        pltpu.VMEM((APW,), jnp.int32),
        pltpu.VMEM((APW,), jnp.int32),
        pltpu.VMEM((32,), jnp.int32),
        pltpu.VMEM((32,), jnp.int32),
    ],
)


GCH = 32
GQ = NROWS // GCH // NW


def _gather_body(x_hbm, stok_hbm, xs_hbm,
                 idx0, idx1, b0, b1, gs0, gs1, ws0, ws1):
    cid = lax.axis_index("c")
    sid = lax.axis_index("s")
    wid = sid * NC + cid
    bufs = [(idx0, b0, gs0, ws0), (idx1, b1, gs1, ws1)]
    gdesc = [None, None]
    wdesc = [None, None]

    def start(q):
        ib, bb, gs, _ = bufs[q % 2]
        if wdesc[q % 2] is not None:
            wdesc[q % 2].wait()
        base = (wid + q * NW) * GCH
        pltpu.sync_copy(stok_hbm.at[pl.ds(base, GCH)], ib)
        for c in range(GCH // 16):
            sl = pl.ds(c * 16, 16)
            ib[sl] = ib[sl] & (T - 1)
        gdesc[q % 2] = pltpu.async_copy(x_hbm.at[ib], bb, gs)

    start(0)
    for q in range(GQ):
        if q + 1 < GQ:
            start(q + 1)
        ib, bb, _, ws = bufs[q % 2]
        gdesc[q % 2].wait()
        base = (wid + q * NW) * GCH
        wdesc[q % 2] = pltpu.async_copy(bb, xs_hbm.at[pl.ds(base, GCH)], ws)
    wdesc[(GQ - 1) % 2].wait()
    wdesc[GQ % 2].wait()


_gather = pl.kernel(
    _gather_body,
    out_type=jax.ShapeDtypeStruct((NROWS, H), jnp.float32),
    mesh=_sc_mesh,
    compiler_params=_sc_params,
    scratch_types=[
        pltpu.VMEM((GCH,), jnp.int32),
        pltpu.VMEM((GCH,), jnp.int32),
        pltpu.VMEM((GCH, H), jnp.float32),
        pltpu.VMEM((GCH, H), jnp.float32),
        pltpu.SemaphoreType.DMA,
        pltpu.SemaphoreType.DMA,
        pltpu.SemaphoreType.DMA,
        pltpu.SemaphoreType.DMA,
    ],
)


def _gemm_body(bexp_ref, act_ref, xs_ref, w1_ref, v1_ref, w2_ref, y_ref):
    b = pl.program_id(0)

    @pl.when(act_ref[b] > 0)
    def _():
        xb = xs_ref[...].astype(jnp.bfloat16)
        w1b = w1_ref[0].astype(jnp.bfloat16)
        v1b = v1_ref[0].astype(jnp.bfloat16)
        w2b = w2_ref[0].astype(jnp.bfloat16)
        h1 = lax.dot_general(
            xb, w1b, (((1,), (1,)), ((), ())), preferred_element_type=jnp.float32
        )
        h2 = lax.dot_general(
            xb, v1b, (((1,), (1,)), ((), ())), preferred_element_type=jnp.float32
        )
        h = (h1 * jax.nn.sigmoid(h1) * h2).astype(jnp.bfloat16)
        y_ref[...] = lax.dot_general(
            h, w2b, (((1,), (0,)), ((), ())), preferred_element_type=jnp.float32
        )


def _gemm(bexp, act, xs, w1, v1, w2):
    grid_spec = pltpu.PrefetchScalarGridSpec(
        num_scalar_prefetch=2,
        grid=(NBLK,),
        in_specs=[
            pl.BlockSpec((RBLK, H), lambda b, be, act: (act[b] * b, 0)),
            pl.BlockSpec((1, F, H), lambda b, be, act: (be[b], 0, 0)),
            pl.BlockSpec((1, F, H), lambda b, be, act: (be[b], 0, 0)),
            pl.BlockSpec((1, F, H), lambda b, be, act: (be[b], 0, 0)),
        ],
        out_specs=pl.BlockSpec((RBLK, H), lambda b, be, act: (b, 0)),
    )
    return pl.pallas_call(
        _gemm_body,
        grid_spec=grid_spec,
        out_shape=jax.ShapeDtypeStruct((NROWS, H), jnp.float32),
    )(bexp, act, xs, w1, v1, w2)


CCH = 16
CQ = T // CCH // NW


def _combine_body(y_hbm, sortpos_hbm, wts_hbm, out_hbm,
                  i0, i1, wb0, wb1, b0, b1, ob, gs0, gs1, osem):
    cid = lax.axis_index("c")
    sid = lax.axis_index("s")
    wid = sid * NC + cid
    lane = _lane16()
    bufs = [(i0, wb0, b0, gs0), (i1, wb1, b1, gs1)]
    gdesc = [None, None]

    def start(q):
        ib, wb, bb, gs = bufs[q % 2]
        t0 = (wid + q * NW) * CCH
        pltpu.sync_copy(sortpos_hbm.at[pl.ds(t0, CCH)], ib.at[pl.ds(0, CCH)])
        pltpu.sync_copy(sortpos_hbm.at[pl.ds(T + t0, CCH)],
                        ib.at[pl.ds(CCH, CCH)])
        pltpu.sync_copy(wts_hbm.at[0, pl.ds(t0, CCH)], wb.at[pl.ds(0, CCH)])
        pltpu.sync_copy(wts_hbm.at[1, pl.ds(t0, CCH)], wb.at[pl.ds(CCH, CCH)])
        gdesc[q % 2] = pltpu.async_copy(y_hbm.at[ib], bb, gs)

    start(0)
    for q in range(CQ):
        if q + 1 < CQ:
            start(q + 1)
        ib, wb, bb, gs = bufs[q % 2]
        gdesc[q % 2].wait()
        w0v = wb[pl.ds(0, 16)]
        w1v = wb[pl.ds(16, 16)]
        s0 = [jnp.sum(jnp.where(lane == i, w0v, 0.0)) for i in range(CCH)]
        s1 = [jnp.sum(jnp.where(lane == i, w1v, 0.0)) for i in range(CCH)]

        def body(c, _):
            sl = pl.ds(c * 16, 16)
            for i in range(CCH):
                ob[i, sl] = bb[i, sl] * s0[i] + bb[CCH + i, sl] * s1[i]
            return 0

        lax.fori_loop(0, H // 16, body, 0)
        t0 = (wid + q * NW) * CCH
        pltpu.sync_copy(ob, out_hbm.at[pl.ds(t0, CCH)])


_combine = pl.kernel(
    _combine_body,
    out_type=jax.ShapeDtypeStruct((T, H), jnp.float32),
    mesh=_sc_mesh,
    compiler_params=_sc_params,
    scratch_types=[
        pltpu.VMEM((2 * CCH,), jnp.int32),
        pltpu.VMEM((2 * CCH,), jnp.int32),
        pltpu.VMEM((2 * CCH,), jnp.float32),
        pltpu.VMEM((2 * CCH,), jnp.float32),
        pltpu.VMEM((2 * CCH, H), jnp.float32),
        pltpu.VMEM((2 * CCH, H), jnp.float32),
        pltpu.VMEM((CCH, H), jnp.float32),
        pltpu.SemaphoreType.DMA,
        pltpu.SemaphoreType.DMA,
        pltpu.SemaphoreType.DMA,
    ],
)


@jax.jit
def kernel(hidden_states, router_w, w1, v1, w2):
    xf = hidden_states.reshape(T, H)
    rw_pad = jnp.zeros((LANES, H), jnp.float32).at[:E].set(router_w)

    eids, wts, hist = _router(xf, rw_pad)
    eids_flat = eids[:TOP_K].reshape(A)
    sortpos, sorted_tok, bexp, act = _dispatch(eids_flat, hist)
    xs = _gather(xf, sorted_tok)
    y = _gemm(bexp[0, :NBLK], act[0, :NBLK], xs, w1, v1, w2)
    out = _combine(y, sortpos, wts[:TOP_K])
    return out.reshape(1, T, H)


# speedup vs baseline: 2.1550x; 1.2154x over previous
import jax
import jax.numpy as jnp
from jax import lax
from jax.experimental import pallas as pl
from jax.experimental.pallas import tpu as pltpu
from jax.experimental.pallas import tpu_sc as plsc

E = 8
TOP_K = 2
H = 1024
F = 1024
T = 2048
A = TOP_K * T
RBLK = 256
NBLK = A // RBLK + E
NROWS = NBLK * RBLK
LANES = 128
NC = 2
NS = 16
NW = NC * NS
APW = A // NW

_sc_mesh = plsc.VectorSubcoreMesh(
    core_axis_name="c", subcore_axis_name="s", num_cores=NC, num_subcores=NS
)
_sc_params = pltpu.CompilerParams(needs_layout_passes=False)


def _lane16():
    return lax.broadcasted_iota(jnp.int32, (16,), 0)


def _router_body(x_ref, rw_ref, eids_ref, wts_ref, hist_ref):
    rw = rw_ref[...]
    x = x_ref[...]
    logits = lax.dot_general(
        rw, x, (((1,), (1,)), ((), ())), preferred_element_type=jnp.float32
    )
    row = lax.broadcasted_iota(jnp.int32, logits.shape, 0)
    neg = jnp.float32(-1e30)
    logits = jnp.where(row < E, logits, neg)
    m = jnp.max(logits, axis=0, keepdims=True)
    ex = jnp.exp(logits - m)
    ex = jnp.where(row < E, ex, 0.0)
    scores = ex / jnp.sum(ex, axis=0, keepdims=True)
    big = jnp.int32(LANES)
    m1 = jnp.max(scores, axis=0, keepdims=True)
    i1 = jnp.min(jnp.where(scores == m1, row, big), axis=0, keepdims=True)
    sc2 = jnp.where(row == i1, neg, scores)
    m2 = jnp.max(sc2, axis=0, keepdims=True)
    i2 = jnp.min(jnp.where(sc2 == m2, row, big), axis=0, keepdims=True)
    denom = m1 + m2
    krow = lax.broadcasted_iota(jnp.int32, (E, T), 0)
    eids_ref[...] = jnp.where(
        krow == 0, jnp.broadcast_to(i1, (E, T)),
        jnp.where(krow == 1, jnp.broadcast_to(i2, (E, T)), 0),
    )
    wts_ref[...] = jnp.where(
        krow == 0, jnp.broadcast_to(m1 / denom, (E, T)),
        jnp.where(krow == 1, jnp.broadcast_to(m2 / denom, (E, T)), 0.0),
    )
    erow = lax.broadcasted_iota(jnp.int32, (E, T), 0)
    oh0 = (erow == jnp.broadcast_to(i1, (E, T))).astype(jnp.float32)
    oh1 = (erow == jnp.broadcast_to(i2, (E, T))).astype(jnp.float32)
    tw = lax.broadcasted_iota(jnp.int32, (T, NS), 0) // APW
    ww = lax.broadcasted_iota(jnp.int32, (T, NS), 1)
    sel = (tw == ww).astype(jnp.float32)
    h0 = lax.dot_general(oh0, sel, (((1,), (0,)), ((), ())),
                         preferred_element_type=jnp.float32)
    h1 = lax.dot_general(oh1, sel, (((1,), (0,)), ((), ())),
                         preferred_element_type=jnp.float32)
    h01 = jnp.concatenate([h0, h1], axis=1).astype(jnp.int32)
    hist_ref[...] = jnp.concatenate(
        [h01, jnp.zeros((E, LANES - NW), jnp.int32)], axis=1
    )


def _router(xf, rw_pad):
    return pl.pallas_call(
        _router_body,
        grid=(1,),
        in_specs=[
            pl.BlockSpec((T, H), lambda i: (0, 0)),
            pl.BlockSpec((LANES, H), lambda i: (0, 0)),
        ],
        out_specs=[
            pl.BlockSpec((E, T), lambda i: (0, 0)),
            pl.BlockSpec((E, T), lambda i: (0, 0)),
            pl.BlockSpec((E, LANES), lambda i: (0, 0)),
        ],
        out_shape=[
            jax.ShapeDtypeStruct((E, T), jnp.int32),
            jax.ShapeDtypeStruct((E, T), jnp.float32),
            jax.ShapeDtypeStruct((E, LANES), jnp.int32),
        ],
    )(xf, rw_pad)


def _dispatch_body(eids_hbm, hist_hbm, sortpos_hbm, bexp_hbm,
                   act_hbm, ev, hb, posb, bexb, actb):
    cid = lax.axis_index("c")
    sid = lax.axis_index("s")
    wid = sid * NC + cid
    lane = _lane16()
    zeros16 = jnp.zeros((16,), jnp.int32)

    pltpu.sync_copy(eids_hbm.at[pl.ds(wid * APW, APW)], ev)
    pltpu.sync_copy(hist_hbm, hb)

    tot_s, base_s = [], []
    for e in range(E):
        he0 = hb[e, pl.ds(0, 16)]
        he1 = hb[e, pl.ds(16, 16)]
        tot_s.append(jnp.sum(he0) + jnp.sum(he1))
        base_s.append(
            jnp.sum(jnp.where(lane < wid, he0, 0))
            + jnp.sum(jnp.where(lane + 16 < wid, he1, 0))
        )
    total = zeros16
    for e in range(E):
        total = total + jnp.where(lane == e, tot_s[e], 0)
    padded = ((total + (RBLK - 1)) >> 8) << 8
    ex_off = plsc.cumsum(padded) - padded

    off_s = [jnp.sum(jnp.where(lane == e, ex_off, 0)) for e in range(E)]
    pad_s = [jnp.sum(jnp.where(lane == e, padded, 0)) for e in range(E)]
    cnt = [off_s[e] + base_s[e] for e in range(E)]

    nr = jnp.sum(padded)
    last_e = jnp.max(jnp.where(padded > 0, lane, 0))
    for v in range(2):
        b = lane + v * 16
        r0 = b * RBLK
        bx = zeros16
        for e in range(E):
            inside = (r0 >= off_s[e]) & (r0 < off_s[e] + pad_s[e])
            bx = bx + jnp.where(inside, e, 0)
        active = r0 < nr
        bexb[pl.ds(v * 16, 16)] = jnp.where(active, bx, last_e)
        actb[pl.ds(v * 16, 16)] = jnp.where(active, 1, 0)
    pltpu.sync_copy(bexb, bexp_hbm.at[wid])
    pltpu.sync_copy(actb, act_hbm.at[wid])

    for j in range(APW // 16):
        evj = ev[pl.ds(j * 16, 16)]
        pos = zeros16
        for e in range(E):
            mask = evj == e
            mi = jnp.where(mask, 1, 0)
            pref = plsc.cumsum(mi) - mi
            pos = jnp.where(mask, cnt[e] + pref, pos)
            cnt[e] = cnt[e] + jnp.sum(mi)
        posb[pl.ds(j * 16, 16)] = pos
    pltpu.sync_copy(posb, sortpos_hbm.at[pl.ds(wid * APW, APW)])


_dispatch = pl.kernel(
    _dispatch_body,
    out_type=[
        jax.ShapeDtypeStruct((A,), jnp.int32),
        jax.ShapeDtypeStruct((NW, 32), jnp.int32),
        jax.ShapeDtypeStruct((NW, 32), jnp.int32),
    ],
    mesh=_sc_mesh,
    compiler_params=_sc_params,
    scratch_types=[
        pltpu.VMEM((APW,), jnp.int32),
        pltpu.VMEM((E, LANES), jnp.int32),
        pltpu.VMEM((APW,), jnp.int32),
        pltpu.VMEM((32,), jnp.int32),
        pltpu.VMEM((32,), jnp.int32),
    ],
)


GCH = 32
GQ = APW // GCH


def _xscatter_body(x_hbm, sortpos_hbm, xs_hbm,
                   idx0, idx1, b0, b1, gs0, gs1, ws0, ws1):
    cid = lax.axis_index("c")
    sid = lax.axis_index("s")
    wid = sid * NC + cid
    bufs = [(idx0, b0, gs0, ws0), (idx1, b1, gs1, ws1)]
    rdesc = [None, None]
    wdesc = [None, None]

    def start(q):
        ib, bb, gs, _ = bufs[q % 2]
        if wdesc[q % 2] is not None:
            wdesc[q % 2].wait()
        a0 = wid * APW + q * GCH
        t0 = pl.multiple_of(a0 & (T - 1), GCH)
        pltpu.sync_copy(sortpos_hbm.at[pl.ds(a0, GCH)], ib)
        rdesc[q % 2] = pltpu.async_copy(x_hbm.at[pl.ds(t0, GCH)], bb, gs)

    start(0)
    for q in range(GQ):
        if q + 1 < GQ:
            start(q + 1)
        ib, bb, _, ws = bufs[q % 2]
        rdesc[q % 2].wait()
        wdesc[q % 2] = pltpu.async_copy(bb, xs_hbm.at[ib], ws)
    wdesc[(GQ - 1) % 2].wait()
    wdesc[GQ % 2].wait()


_xscatter = pl.kernel(
    _xscatter_body,
    out_type=jax.ShapeDtypeStruct((NROWS, H), jnp.float32),
    mesh=_sc_mesh,
    compiler_params=_sc_params,
    scratch_types=[
        pltpu.VMEM((GCH,), jnp.int32),
        pltpu.VMEM((GCH,), jnp.int32),
        pltpu.VMEM((GCH, H), jnp.float32),
        pltpu.VMEM((GCH, H), jnp.float32),
        pltpu.SemaphoreType.DMA,
        pltpu.SemaphoreType.DMA,
        pltpu.SemaphoreType.DMA,
        pltpu.SemaphoreType.DMA,
    ],
)


def _gemm_body(bexp_ref, act_ref, xs_ref, w1_ref, v1_ref, w2_ref, y_ref):
    b = pl.program_id(0)

    @pl.when(act_ref[b] > 0)
    def _():
        xb = xs_ref[...].astype(jnp.bfloat16)
        w1b = w1_ref[0].astype(jnp.bfloat16)
        v1b = v1_ref[0].astype(jnp.bfloat16)
        w2b = w2_ref[0].astype(jnp.bfloat16)
        h1 = lax.dot_general(
            xb, w1b, (((1,), (1,)), ((), ())), preferred_element_type=jnp.float32
        )
        h2 = lax.dot_general(
            xb, v1b, (((1,), (1,)), ((), ())), preferred_element_type=jnp.float32
        )
        h = (h1 * jax.nn.sigmoid(h1) * h2).astype(jnp.bfloat16)
        y_ref[...] = lax.dot_general(
            h, w2b, (((1,), (0,)), ((), ())), preferred_element_type=jnp.float32
        )


def _gemm(bexp, act, xs, w1, v1, w2):
    grid_spec = pltpu.PrefetchScalarGridSpec(
        num_scalar_prefetch=2,
        grid=(NBLK,),
        in_specs=[
            pl.BlockSpec((RBLK, H), lambda b, be, act: (act[b] * b, 0)),
            pl.BlockSpec((1, F, H), lambda b, be, act: (be[b], 0, 0)),
            pl.BlockSpec((1, F, H), lambda b, be, act: (be[b], 0, 0)),
            pl.BlockSpec((1, F, H), lambda b, be, act: (be[b], 0, 0)),
        ],
        out_specs=pl.BlockSpec((RBLK, H), lambda b, be, act: (b, 0)),
    )
    return pl.pallas_call(
        _gemm_body,
        grid_spec=grid_spec,
        out_shape=jax.ShapeDtypeStruct((NROWS, H), jnp.float32),
    )(bexp, act, xs, w1, v1, w2)


CCH = 16
CQ = T // CCH // NW


def _combine_body(y_hbm, sortpos_hbm, wts_hbm, out_hbm,
                  i0, i1, wb0, wb1, b0, b1, ob, gs0, gs1, osem):
    cid = lax.axis_index("c")
    sid = lax.axis_index("s")
    wid = sid * NC + cid
    lane = _lane16()
    bufs = [(i0, wb0, b0, gs0), (i1, wb1, b1, gs1)]
    gdesc = [None, None]

    def start(q):
        ib, wb, bb, gs = bufs[q % 2]
        t0 = (wid + q * NW) * CCH
        pltpu.sync_copy(sortpos_hbm.at[pl.ds(t0, CCH)], ib.at[pl.ds(0, CCH)])
        pltpu.sync_copy(sortpos_hbm.at[pl.ds(T + t0, CCH)],
                        ib.at[pl.ds(CCH, CCH)])
        pltpu.sync_copy(wts_hbm.at[0, pl.ds(t0, CCH)], wb.at[pl.ds(0, CCH)])
        pltpu.sync_copy(wts_hbm.at[1, pl.ds(t0, CCH)], wb.at[pl.ds(CCH, CCH)])
        gdesc[q % 2] = pltpu.async_copy(y_hbm.at[ib], bb, gs)

    start(0)
    for q in range(CQ):
        if q + 1 < CQ:
            start(q + 1)
        ib, wb, bb, gs = bufs[q % 2]
        gdesc[q % 2].wait()
        w0v = wb[pl.ds(0, 16)]
        w1v = wb[pl.ds(16, 16)]
        s0 = [jnp.sum(jnp.where(lane == i, w0v, 0.0)) for i in range(CCH)]
        s1 = [jnp.sum(jnp.where(lane == i, w1v, 0.0)) for i in range(CCH)]

        def body(c, _):
            sl = pl.ds(c * 16, 16)
            for i in range(CCH):
                ob[i, sl] = bb[i, sl] * s0[i] + bb[CCH + i, sl] * s1[i]
            return 0

        lax.fori_loop(0, H // 16, body, 0)
        t0 = (wid + q * NW) * CCH
        pltpu.sync_copy(ob, out_hbm.at[pl.ds(t0, CCH)])


_combine = pl.kernel(
    _combine_body,
    out_type=jax.ShapeDtypeStruct((T, H), jnp.float32),
    mesh=_sc_mesh,
    compiler_params=_sc_params,
    scratch_types=[
        pltpu.VMEM((2 * CCH,), jnp.int32),
        pltpu.VMEM((2 * CCH,), jnp.int32),
        pltpu.VMEM((2 * CCH,), jnp.float32),
        pltpu.VMEM((2 * CCH,), jnp.float32),
        pltpu.VMEM((2 * CCH, H), jnp.float32),
        pltpu.VMEM((2 * CCH, H), jnp.float32),
        pltpu.VMEM((CCH, H), jnp.float32),
        pltpu.SemaphoreType.DMA,
        pltpu.SemaphoreType.DMA,
        pltpu.SemaphoreType.DMA,
    ],
)


@jax.jit
def kernel(hidden_states, router_w, w1, v1, w2):
    xf = hidden_states.reshape(T, H)
    rw_pad = jnp.zeros((LANES, H), jnp.float32).at[:E].set(router_w)

    eids, wts, hist = _router(xf, rw_pad)
    eids_flat = eids[:TOP_K].reshape(A)
    sortpos, bexp, act = _dispatch(eids_flat, hist)
    xs = _xscatter(xf, sortpos)
    y = _gemm(bexp[0, :NBLK], act[0, :NBLK], xs, w1, v1, w2)
    out = _combine(y, sortpos, wts[:TOP_K])
    return out.reshape(1, T, H)
